# baseline (speedup 1.0000x reference)
import functools

import jax
import jax.numpy as jnp
from jax import lax
from jax.experimental import pallas as pl
from jax.experimental.pallas import tpu as pltpu
from jax.experimental.pallas import tpu_sc as plsc

NC = 2
NS = 16
NW = NC * NS
NTC = 2
LANES = 16
VCHUNK = 6272
NCHUNK = 16
FSTRIDE = VCHUNK * NCHUNK
NSLOT = 4
GROWS = 64


def _repack_kernel(F, CARD2, E):
    fields_per_core = F // NTC
    mesh = pltpu.create_tensorcore_mesh("core", num_cores=NTC)

    @functools.partial(
        pl.kernel,
        out_type=jax.ShapeDtypeStruct((F * FSTRIDE, 128), jnp.float32),
        mesh=mesh,
    )
    def kern(x_hbm, o_hbm):
        core = lax.axis_index("core")
        fbase = core * fields_per_core

        def body(x_ref, o_ref):
            o_ref[:, 0:E] = x_ref[0].T

        pltpu.emit_pipeline(
            body,
            grid=(fields_per_core, NCHUNK),
            in_specs=[pl.BlockSpec((1, E, VCHUNK),
                                   lambda f, c: (fbase + f, 0, c))],
            out_specs=[pl.BlockSpec((VCHUNK, 128),
                                    lambda f, c: ((fbase + f) * NCHUNK + c,
                                                  0))],
        )(x_hbm, o_hbm)

    return kern


def _sc_kernel(B, F, CARD2, E):
    rows_per_w = B // NW
    n_idx = rows_per_w // 128
    n_slices = rows_per_w // GROWS
    total_slices = F * n_slices
    mesh = plsc.VectorSubcoreMesh(core_axis_name="c", subcore_axis_name="s",
                                  num_cores=NC, num_subcores=NS)

    @functools.partial(
        pl.kernel,
        out_type=jax.ShapeDtypeStruct((B, 128), jnp.float32),
        mesh=mesh,
        compiler_params=pltpu.CompilerParams(use_tc_tiling_on_sc=True),
        scratch_types=(
            [pltpu.VMEM((F, n_idx, 128), jnp.int32),
             pltpu.VMEM((n_slices, GROWS), jnp.int32),
             pltpu.VMEM_SHARED((NS * rows_per_w, 128), jnp.float32),
             pltpu.VMEM((NSLOT, GROWS, 128), jnp.float32)]
            + [pltpu.SemaphoreType.DMA] * (2 * NSLOT)
        ),
    )
    def kern(tab_hbm, idx_hbm, out_hbm, idx_v, oidx_v, acc_sh, buf_v, *sems):
        gsem = sems[:NSLOT]
        ssem = sems[NSLOT:]
        sid = lax.axis_index("s")
        wid = sid * NC + lax.axis_index("c")
        base = sid * rows_per_w

        for m in range(n_idx):
            pltpu.sync_copy(
                idx_hbm.at[:, pl.ds(wid * rows_per_w + m * 128, 128)],
                idx_v.at[:, m, :])

        @pl.loop(0, F)
        def _(f):
            off = f * FSTRIDE + 1

            @pl.loop(0, n_idx)
            def _(m):
                @pl.loop(0, 128 // LANES)
                def _(k):
                    sl = pl.ds(k * LANES, LANES)
                    idx_v[f, m, sl] = idx_v[f, m, sl] + off

        @pl.loop(0, n_slices)
        def _(m):
            @pl.loop(0, GROWS // LANES)
            def _(k):
                oidx_v[m, pl.ds(k * LANES, LANES)] = (
                    lax.iota(jnp.int32, LANES)
                    + (base + m * GROWS + k * LANES))

        zeros16 = jnp.zeros((LANES,), jnp.float32)

        @pl.loop(0, GROWS)
        def _(r):
            @pl.loop(0, 128 // LANES)
            def _(k):
                buf_v[0, r, pl.ds(k * LANES, LANES)] = zeros16
        for m in range(n_slices):
            pltpu.sync_copy(buf_v.at[0],
                            acc_sh.at[pl.ds(base + m * GROWS, GROWS)])

        def slice_refs(s):
            f = lax.div(s, n_slices)
            sub = lax.rem(s, n_slices)
            m = lax.div(sub, n_slices // n_idx)
            h = lax.rem(sub, n_slices // n_idx)
            return idx_v.at[f, m, pl.ds(h * GROWS, GROWS)], oidx_v.at[sub]

        for t in range(NSLOT):
            src, _ = slice_refs(jnp.int32(t))
            pltpu.async_copy(tab_hbm.at[src], buf_v.at[t], gsem[t])

        @pl.loop(0, total_slices, step=NSLOT)
        def _(j):
            for t in range(NSLOT):
                src, dst = slice_refs(j + t)
                pltpu.make_async_copy(tab_hbm.at[src], buf_v.at[t],
                                      gsem[t]).wait()
                pltpu.async_copy(buf_v.at[t], acc_sh.at[dst], ssem[t],
                                 add=True)
            for t in range(NSLOT):
                _, dst = slice_refs(j + t)
                pltpu.make_async_copy(buf_v.at[t], acc_sh.at[dst],
                                      ssem[t]).wait()

                @pl.when(j + NSLOT + t < total_slices)
                def _():
                    src, _ = slice_refs(j + NSLOT + t)
                    pltpu.async_copy(tab_hbm.at[src], buf_v.at[t], gsem[t])

        pltpu.sync_copy(acc_sh.at[pl.ds(base, rows_per_w)],
                        out_hbm.at[pl.ds(wid * rows_per_w, rows_per_w)])

    return kern


def kernel(indices, tables):
    F, CARD2, E = tables.shape
    B = indices.shape[0]
    tab_t = jnp.transpose(tables, (0, 2, 1))
    idx_t = jnp.transpose(indices)
    packed = _repack_kernel(F, CARD2, E)(tab_t)
    wide = _sc_kernel(B, F, CARD2, E)(packed, idx_t)
    return wide[:, :E]


# speedup vs baseline: 1.2361x; 1.2361x over previous
import functools

import jax
import jax.numpy as jnp
from jax import lax
from jax.experimental import pallas as pl
from jax.experimental.pallas import tpu as pltpu
from jax.experimental.pallas import tpu_sc as plsc

NC = 2
NS = 16
NW = NC * NS
NTC = 2
LANES = 16
VCHUNK = 25088
NCHUNK = 4
FSTRIDE = VCHUNK * NCHUNK
NSLOT = 4
GROWS = 64


def _repack_kernel(F, CARD2, E):
    fields_per_core = F // NTC
    mesh = pltpu.create_tensorcore_mesh("core", num_cores=NTC)

    @functools.partial(
        pl.kernel,
        out_type=jax.ShapeDtypeStruct((F * FSTRIDE, 128), jnp.float32),
        mesh=mesh,
    )
    def kern(x_hbm, o_hbm):
        core = lax.axis_index("core")
        fbase = core * fields_per_core

        def body(x_ref, o_ref):
            o_ref[:, 0:E] = x_ref[0].T

        pltpu.emit_pipeline(
            body,
            grid=(fields_per_core, NCHUNK),
            in_specs=[pl.BlockSpec((1, E, VCHUNK),
                                   lambda f, c: (fbase + f, 0, c))],
            out_specs=[pl.BlockSpec((VCHUNK, 128),
                                    lambda f, c: ((fbase + f) * NCHUNK + c,
                                                  0))],
        )(x_hbm, o_hbm)

    return kern


def _sc_kernel(B, F, CARD2, E):
    rows_per_w = B // NW
    n_idx = rows_per_w // 128
    n_slices = rows_per_w // GROWS
    total_slices = F * n_slices
    mesh = plsc.VectorSubcoreMesh(core_axis_name="c", subcore_axis_name="s",
                                  num_cores=NC, num_subcores=NS)

    @functools.partial(
        pl.kernel,
        out_type=jax.ShapeDtypeStruct((B, 128), jnp.float32),
        mesh=mesh,
        compiler_params=pltpu.CompilerParams(use_tc_tiling_on_sc=True),
        scratch_types=(
            [pltpu.VMEM((F, n_idx, 128), jnp.int32),
             pltpu.VMEM((n_slices, GROWS), jnp.int32),
             pltpu.VMEM_SHARED((NS * rows_per_w, 128), jnp.float32),
             pltpu.VMEM((NSLOT, GROWS, 128), jnp.float32)]
            + [pltpu.SemaphoreType.DMA] * (2 * NSLOT)
        ),
    )
    def kern(tab_hbm, idx_hbm, out_hbm, idx_v, oidx_v, acc_sh, buf_v, *sems):
        gsem = sems[:NSLOT]
        ssem = sems[NSLOT:]
        sid = lax.axis_index("s")
        wid = sid * NC + lax.axis_index("c")
        base = sid * rows_per_w

        for m in range(n_idx):
            pltpu.sync_copy(
                idx_hbm.at[:, pl.ds(wid * rows_per_w + m * 128, 128)],
                idx_v.at[:, m, :])

        @pl.loop(0, F)
        def _(f):
            off = f * FSTRIDE + 1

            @pl.loop(0, n_idx)
            def _(m):
                @pl.loop(0, 128 // LANES)
                def _(k):
                    sl = pl.ds(k * LANES, LANES)
                    idx_v[f, m, sl] = idx_v[f, m, sl] + off

        @pl.loop(0, n_slices)
        def _(m):
            @pl.loop(0, GROWS // LANES)
            def _(k):
                oidx_v[m, pl.ds(k * LANES, LANES)] = (
                    lax.iota(jnp.int32, LANES)
                    + (base + m * GROWS + k * LANES))

        zeros16 = jnp.zeros((LANES,), jnp.float32)

        @pl.loop(0, GROWS)
        def _(r):
            @pl.loop(0, 128 // LANES)
            def _(k):
                buf_v[0, r, pl.ds(k * LANES, LANES)] = zeros16
        for m in range(n_slices):
            pltpu.sync_copy(buf_v.at[0],
                            acc_sh.at[pl.ds(base + m * GROWS, GROWS)])

        def slice_refs(s):
            f = lax.div(s, n_slices)
            sub = lax.rem(s, n_slices)
            m = lax.div(sub, n_slices // n_idx)
            h = lax.rem(sub, n_slices // n_idx)
            return idx_v.at[f, m, pl.ds(h * GROWS, GROWS)], oidx_v.at[sub]

        for t in range(NSLOT):
            src, _ = slice_refs(jnp.int32(t))
            pltpu.async_copy(tab_hbm.at[src], buf_v.at[t], gsem[t])

        @pl.loop(0, total_slices, step=NSLOT)
        def _(j):
            for t in range(NSLOT):
                src, dst = slice_refs(j + t)
                pltpu.make_async_copy(tab_hbm.at[src], buf_v.at[t],
                                      gsem[t]).wait()
                pltpu.async_copy(buf_v.at[t], acc_sh.at[dst], ssem[t],
                                 add=True)
            for t in range(NSLOT):
                _, dst = slice_refs(j + t)
                pltpu.make_async_copy(buf_v.at[t], acc_sh.at[dst],
                                      ssem[t]).wait()

                @pl.when(j + NSLOT + t < total_slices)
                def _():
                    src, _ = slice_refs(j + NSLOT + t)
                    pltpu.async_copy(tab_hbm.at[src], buf_v.at[t], gsem[t])

        pltpu.sync_copy(acc_sh.at[pl.ds(base, rows_per_w)],
                        out_hbm.at[pl.ds(wid * rows_per_w, rows_per_w)])

    return kern


def kernel(indices, tables):
    F, CARD2, E = tables.shape
    B = indices.shape[0]
    tab_t = jnp.transpose(tables, (0, 2, 1))
    idx_t = jnp.transpose(indices)
    packed = _repack_kernel(F, CARD2, E)(tab_t)
    wide = _sc_kernel(B, F, CARD2, E)(packed, idx_t)
    return wide[:, :E]
